# baseline (speedup 1.0000x reference)
import functools

import jax
import jax.numpy as jnp
from jax.experimental import pallas as pl
from jax.experimental.pallas import tpu as pltpu

DIM = 768
HEADS = 12
DIM_HEAD = 64
EXPERTS = 8
HIDDEN = DIM * 4
TOP_K = 2
T = 2048
SI = 1024
BM = 128
S = TOP_K * T + EXPERTS * BM
NBLK = S // BM


def _bdot(a, b, dn=None):
    a16 = a.astype(jnp.bfloat16)
    b16 = b.astype(jnp.bfloat16)
    if dn is None:
        return jnp.dot(a16, b16, preferred_element_type=jnp.float32)
    return jax.lax.dot_general(a16, b16, dn,
                               preferred_element_type=jnp.float32)


def _ln(x, g):
    mu = jnp.mean(x, axis=-1, keepdims=True)
    xc = x - mu
    var = jnp.mean(xc * xc, axis=-1, keepdims=True)
    return xc / jnp.sqrt(var + 1e-5) * g


def _v_body(img_ref, wv_ref, v_ref):
    v_ref[...] = _bdot(img_ref[...], wv_ref[...])


def _v_proj(img, Wv):
    return pl.pallas_call(
        _v_body,
        grid=(4,),
        in_specs=[
            pl.BlockSpec((SI // 4, DIM), lambda i: (i, 0)),
            pl.BlockSpec((DIM, DIM), lambda i: (0, 0)),
        ],
        out_specs=pl.BlockSpec((SI // 4, DIM), lambda i: (i, 0)),
        out_shape=jax.ShapeDtypeStruct((SI, DIM), jnp.float32),
    )(img, Wv)


def _kt_body(wkt_ref, imgT_ref, kt_ref):
    kt_ref[...] = _bdot(wkt_ref[...], imgT_ref[...])


def _kt_proj(WkT, imgT):
    return pl.pallas_call(
        _kt_body,
        grid=(4,),
        in_specs=[
            pl.BlockSpec((DIM, DIM), lambda i: (0, 0)),
            pl.BlockSpec((DIM, SI // 4), lambda i: (0, i)),
        ],
        out_specs=pl.BlockSpec((DIM, SI // 4), lambda i: (0, i)),
        out_shape=jax.ShapeDtypeStruct((DIM, SI), jnp.float32),
    )(WkT, imgT)


def _attn_body(text_ref, kt_ref, v_ref, lnq_ref, wq_ref, nk_ref, nv_ref,
               wo_ref, lno_ref, gw_ref, act_ref, logits_ref):
    x = text_ref[...]
    xn = _ln(x, lnq_ref[...])
    scale = DIM_HEAD ** -0.5
    q = _bdot(xn, wq_ref[...]) * scale
    outs = []
    for h in range(HEADS):
        sl = slice(h * DIM_HEAD, (h + 1) * DIM_HEAD)
        q_h = q[:, sl]
        kt_h = kt_ref[sl, :]
        v_h = v_ref[:, sl]
        s = _bdot(q_h, kt_h)
        nl = _bdot(q_h, nk_ref[...])
        m = jnp.maximum(jnp.max(s, axis=1, keepdims=True), nl)
        p = jnp.exp(s - m)
        pn = jnp.exp(nl - m)
        den = jnp.sum(p, axis=1, keepdims=True) + pn
        attn = p / den
        attn_n = (pn / den).astype(jnp.bfloat16).astype(jnp.float32)
        nv16 = nv_ref[...].astype(jnp.bfloat16).astype(jnp.float32)
        o = _bdot(attn, v_h) + attn_n * nv16
        outs.append(o)
    out = jnp.concatenate(outs, axis=1)
    att = _ln(_bdot(out, wo_ref[...]), lno_ref[...])
    a = jnp.tanh(att) + x
    act_ref[...] = a
    logits_ref[...] = _bdot(a, gw_ref[...])


def _attention(text, kt, v, ln_q_g, Wq, null_k, null_v, Wo, ln_out_g, gate_W):
    BQ = 256
    return pl.pallas_call(
        _attn_body,
        grid=(T // BQ,),
        in_specs=[
            pl.BlockSpec((BQ, DIM), lambda i: (i, 0)),
            pl.BlockSpec((DIM, SI), lambda i: (0, 0)),
            pl.BlockSpec((SI, DIM), lambda i: (0, 0)),
            pl.BlockSpec((1, DIM), lambda i: (0, 0)),
            pl.BlockSpec((DIM, DIM), lambda i: (0, 0)),
            pl.BlockSpec((DIM_HEAD, 1), lambda i: (0, 0)),
            pl.BlockSpec((1, DIM_HEAD), lambda i: (0, 0)),
            pl.BlockSpec((DIM, DIM), lambda i: (0, 0)),
            pl.BlockSpec((1, DIM), lambda i: (0, 0)),
            pl.BlockSpec((DIM, EXPERTS), lambda i: (0, 0)),
        ],
        out_specs=[
            pl.BlockSpec((BQ, DIM), lambda i: (i, 0)),
            pl.BlockSpec((BQ, EXPERTS), lambda i: (i, 0)),
        ],
        out_shape=[
            jax.ShapeDtypeStruct((T, DIM), jnp.float32),
            jax.ShapeDtypeStruct((T, EXPERTS), jnp.float32),
        ],
    )(text, kt, v, ln_q_g, Wq, null_k, null_v, Wo, ln_out_g, gate_W)


def _dispatch(logits):
    gates = jax.nn.softmax(logits, axis=-1)
    topv, topi = jax.lax.top_k(gates, TOP_K)
    topv = topv / (jnp.sum(topv, axis=-1, keepdims=True) + 1e-9)
    e_all = jnp.concatenate([topi[:, 0], topi[:, 1]]).astype(jnp.int32)
    w_all = jnp.concatenate([topv[:, 0], topv[:, 1]])
    tid_all = jnp.concatenate([jnp.arange(T, dtype=jnp.int32)] * 2)
    counts = jnp.sum(jax.nn.one_hot(e_all, EXPERTS, dtype=jnp.int32), axis=0)
    tight_off = jnp.cumsum(counts) - counts
    padded = ((counts + BM - 1) // BM) * BM
    pad_off = jnp.cumsum(padded) - padded
    order = jnp.argsort(e_all, stable=True)
    j = jnp.arange(2 * T)
    slot_sorted = pad_off[e_all[order]] + (j - tight_off[e_all[order]])
    slot = jnp.zeros((2 * T,), jnp.int32).at[order].set(
        slot_sorted.astype(jnp.int32))
    sorted_tid = jnp.zeros((S,), jnp.int32).at[slot].set(tid_all)
    sorted_w = jnp.zeros((S,), jnp.float32).at[slot].set(w_all)
    block_expert = (jnp.searchsorted(
        pad_off, jnp.arange(NBLK, dtype=jnp.int32) * BM, side='right')
        .astype(jnp.int32) - 1)
    block_expert = jnp.clip(block_expert, 0, EXPERTS - 1)
    return sorted_tid, sorted_w, slot[:T], slot[T:], block_expert


def _ffn_body(be_ref, x_ref, w1_ref, w2_ref, sw_ref, out_ref):
    del be_ref
    h = jax.nn.gelu(_bdot(x_ref[...], w1_ref[0]))
    o = _bdot(h, w2_ref[0])
    out_ref[...] = o * sw_ref[0, 0][:, None]


def _ffn(block_expert, x_sorted, expert_W1, expert_W2, sorted_w3):
    grid_spec = pltpu.PrefetchScalarGridSpec(
        num_scalar_prefetch=1,
        grid=(NBLK,),
        in_specs=[
            pl.BlockSpec((BM, DIM), lambda b, be: (b, 0)),
            pl.BlockSpec((1, DIM, HIDDEN), lambda b, be: (be[b], 0, 0)),
            pl.BlockSpec((1, HIDDEN, DIM), lambda b, be: (be[b], 0, 0)),
            pl.BlockSpec((1, 1, BM), lambda b, be: (b, 0, 0)),
        ],
        out_specs=pl.BlockSpec((BM, DIM), lambda b, be: (b, 0)),
    )
    return pl.pallas_call(
        _ffn_body,
        grid_spec=grid_spec,
        out_shape=jax.ShapeDtypeStruct((S, DIM), jnp.float32),
    )(block_expert, x_sorted, expert_W1, expert_W2, sorted_w3)


def _combine_body(g0_ref, g1_ref, act_ref, out_ref):
    out_ref[...] = jnp.tanh(g0_ref[...] + g1_ref[...] + act_ref[...])


def _combine(g0, g1, activated):
    BQ = 256
    return pl.pallas_call(
        _combine_body,
        grid=(T // BQ,),
        in_specs=[pl.BlockSpec((BQ, DIM), lambda i: (i, 0))] * 3,
        out_specs=pl.BlockSpec((BQ, DIM), lambda i: (i, 0)),
        out_shape=jax.ShapeDtypeStruct((T, DIM), jnp.float32),
    )(g0, g1, activated)


def kernel(text, img, ln_q_g, Wq, Wkv, null_k, null_v, Wo, ln_out_g,
           gate_W, expert_W1, expert_W2):
    B = text.shape[0]
    text2 = text.reshape(T, DIM)
    img2 = img.reshape(SI, DIM)

    v = _v_proj(img2, Wkv[:, DIM:])
    kt = _kt_proj(Wkv[:, :DIM].T, img2.T)
    activated, logits = _attention(
        text2, kt, v, ln_q_g.reshape(1, DIM), Wq,
        null_k.reshape(DIM_HEAD, 1), null_v.reshape(1, DIM_HEAD), Wo,
        ln_out_g.reshape(1, DIM), gate_W)

    sorted_tid, sorted_w, pos0, pos1, block_expert = _dispatch(logits)
    x_sorted = jnp.take(activated, sorted_tid, axis=0)
    out_sorted = _ffn(block_expert, x_sorted, expert_W1, expert_W2,
                      sorted_w.reshape(NBLK, 1, BM))
    g0 = jnp.take(out_sorted, pos0, axis=0)
    g1 = jnp.take(out_sorted, pos1, axis=0)
    out = _combine(g0, g1, activated)
    return out.reshape(B, T, DIM)
